# Initial kernel scaffold; baseline (speedup 1.0000x reference)
import jax
import jax.numpy as jnp
from jax.experimental import pallas as pl


def kernel(input, lengths, fsa_tensor):
    raise NotImplementedError("write your pallas kernel here")


# speedup vs baseline: 19.8629x; 19.8629x over previous
import functools

import jax
import jax.numpy as jnp
from jax import lax
from jax.experimental import pallas as pl
from jax.experimental.pallas import tpu as pltpu
from jax.experimental.pallas import tpu_sc as plsc

V, S, B, L = 100000, 32, 1024, 50
NUM_WORKERS = 32
BPW = B // NUM_WORKERS


def _sc_kernel(inp_hbm, table_hbm, out_hbm, idx_v, gbuf, outb, sem):
    nc = 2
    wid = lax.axis_index("s") * nc + lax.axis_index("c")
    base = wid * BPW

    pltpu.sync_copy(inp_hbm.at[pl.ds(base, BPW)], idx_v)

    def body_b(b, _):
        pltpu.async_copy(table_hbm.at[idx_v.at[b]], gbuf, sem).wait()

        a0 = jnp.clip(gbuf[0, pl.ds(0, 16)], -10.0, 10.0)
        a1 = jnp.clip(gbuf[0, pl.ds(16, 16)], -10.0, 10.0)
        outb[0, pl.ds(0, 16)] = a0
        outb[0, pl.ds(16, 16)] = a1

        def body_t(t, carry):
            h0, h1 = carry
            acc0 = jnp.zeros((16,), jnp.float32)
            acc1 = jnp.zeros((16,), jnp.float32)
            for s in range(16):
                hs = h0[s]
                acc0 = acc0 + hs * gbuf[t, pl.ds(s * S, 16)]
                acc1 = acc1 + hs * gbuf[t, pl.ds(s * S + 16, 16)]
            for s in range(16):
                hs = h1[s]
                acc0 = acc0 + hs * gbuf[t, pl.ds((s + 16) * S, 16)]
                acc1 = acc1 + hs * gbuf[t, pl.ds((s + 16) * S + 16, 16)]
            acc0 = jnp.clip(acc0, -10.0, 10.0)
            acc1 = jnp.clip(acc1, -10.0, 10.0)
            outb[t, pl.ds(0, 16)] = acc0
            outb[t, pl.ds(16, 16)] = acc1
            return (acc0, acc1)

        lax.fori_loop(1, L, body_t, (a0, a1))
        pltpu.sync_copy(outb, out_hbm.at[base + b])
        return 0

    lax.fori_loop(0, BPW, body_b, 0)


@jax.jit
def _run(inp, table2d):
    mesh = plsc.VectorSubcoreMesh(core_axis_name="c", subcore_axis_name="s")
    f = functools.partial(
        pl.kernel,
        mesh=mesh,
        out_type=jax.ShapeDtypeStruct((B, L, S), jnp.float32),
        scratch_types=[
            pltpu.VMEM((BPW, L), jnp.int32),
            pltpu.VMEM((L, S * S), jnp.float32),
            pltpu.VMEM((L, S), jnp.float32),
            pltpu.SemaphoreType.DMA,
        ],
    )(_sc_kernel)
    return f(inp, table2d)


def kernel(input, lengths, fsa_tensor):
    del lengths
    table2d = fsa_tensor.reshape(V, S * S)
    return _run(input.astype(jnp.int32), table2d)


# speedup vs baseline: 22.4939x; 1.1325x over previous
import functools

import jax
import jax.numpy as jnp
from jax import lax
from jax.experimental import pallas as pl
from jax.experimental.pallas import tpu as pltpu
from jax.experimental.pallas import tpu_sc as plsc

V, S, B, L = 100000, 32, 1024, 50
NUM_WORKERS = 32
BPW = B // NUM_WORKERS
CHUNK = L // 2
NCHAIN = 4


def _rnn_chunk(gbuf, outb, t0, lo, carry):
    def body_t(tl, carry):
        h0, h1 = carry
        p0 = [None] * NCHAIN
        p1 = [None] * NCHAIN
        for s in range(S):
            hs = h0[s] if s < 16 else h1[s - 16]
            c = s % NCHAIN
            m0 = hs * gbuf[tl, pl.ds(s * S, 16)]
            m1 = hs * gbuf[tl, pl.ds(s * S + 16, 16)]
            p0[c] = m0 if p0[c] is None else p0[c] + m0
            p1[c] = m1 if p1[c] is None else p1[c] + m1
        acc0 = (p0[0] + p0[1]) + (p0[2] + p0[3])
        acc1 = (p1[0] + p1[1]) + (p1[2] + p1[3])
        acc0 = jnp.clip(acc0, -10.0, 10.0)
        acc1 = jnp.clip(acc1, -10.0, 10.0)
        outb[t0 + tl, pl.ds(0, 16)] = acc0
        outb[t0 + tl, pl.ds(16, 16)] = acc1
        return (acc0, acc1)

    return lax.fori_loop(lo, CHUNK, body_t, carry)


def _sc_kernel(inp_hbm, table_hbm, out_hbm,
               idx_v, gb0, gb1, ob0, ob1, sg0, sg1, so0, so1):
    nc = 2
    wid = lax.axis_index("s") * nc + lax.axis_index("c")
    base = wid * BPW

    pltpu.sync_copy(inp_hbm.at[pl.ds(base, BPW)], idx_v)

    pltpu.async_copy(table_hbm.at[idx_v.at[0, 0, pl.ds(0, CHUNK)]], gb0, sg0)

    def body_pair(i, _):
        for k in range(2):
            b = i * 2 + k
            ocur, socur = (ob0, so0) if k == 0 else (ob1, so1)

            pltpu.async_copy(
                table_hbm.at[idx_v.at[b, 1, pl.ds(0, CHUNK)]], gb1, sg1)

            @pl.when(b >= 2)
            def _():
                pltpu.make_async_copy(
                    ocur, out_hbm.at[base + b - 2], socur).wait()

            pltpu.make_async_copy(
                table_hbm.at[idx_v.at[b, 0, pl.ds(0, CHUNK)]], gb0, sg0).wait()

            a0 = jnp.clip(gb0[0, pl.ds(0, 16)], -10.0, 10.0)
            a1 = jnp.clip(gb0[0, pl.ds(16, 16)], -10.0, 10.0)
            ocur[0, pl.ds(0, 16)] = a0
            ocur[0, pl.ds(16, 16)] = a1
            carry = _rnn_chunk(gb0, ocur, 0, 1, (a0, a1))

            @pl.when(b + 1 < BPW)
            def _():
                pltpu.async_copy(
                    table_hbm.at[idx_v.at[b + 1, 0, pl.ds(0, CHUNK)]], gb0, sg0)

            pltpu.make_async_copy(
                table_hbm.at[idx_v.at[b, 1, pl.ds(0, CHUNK)]], gb1, sg1).wait()
            _rnn_chunk(gb1, ocur, CHUNK, 0, carry)

            pltpu.async_copy(ocur, out_hbm.at[base + b], socur)
        return 0

    lax.fori_loop(0, BPW // 2, body_pair, 0)
    pltpu.make_async_copy(ob0, out_hbm.at[base + BPW - 2], so0).wait()
    pltpu.make_async_copy(ob1, out_hbm.at[base + BPW - 1], so1).wait()


@jax.jit
def _run(inp, table2d):
    mesh = plsc.VectorSubcoreMesh(core_axis_name="c", subcore_axis_name="s")
    f = functools.partial(
        pl.kernel,
        mesh=mesh,
        out_type=jax.ShapeDtypeStruct((B, L, S), jnp.float32),
        scratch_types=[
            pltpu.VMEM((BPW, 2, 32), jnp.int32),
            pltpu.VMEM((CHUNK, S * S), jnp.float32),
            pltpu.VMEM((CHUNK, S * S), jnp.float32),
            pltpu.VMEM((L, S), jnp.float32),
            pltpu.VMEM((L, S), jnp.float32),
            pltpu.SemaphoreType.DMA,
            pltpu.SemaphoreType.DMA,
            pltpu.SemaphoreType.DMA,
            pltpu.SemaphoreType.DMA,
        ],
    )(_sc_kernel)
    return f(inp, table2d)


def kernel(input, lengths, fsa_tensor):
    del lengths
    table2d = fsa_tensor.reshape(V, S * S)
    inp = input.astype(jnp.int32).reshape(B, 2, CHUNK)
    inp = jnp.pad(inp, ((0, 0), (0, 0), (0, 32 - CHUNK)))
    return _run(inp, table2d)


# speedup vs baseline: 23.0754x; 1.0259x over previous
import functools

import jax
import jax.numpy as jnp
from jax import lax
from jax.experimental import pallas as pl
from jax.experimental.pallas import tpu as pltpu
from jax.experimental.pallas import tpu_sc as plsc

V, S, B, L = 100000, 32, 1024, 50
NUM_WORKERS = 32
BPW = B // NUM_WORKERS
CHUNK = L // 2
NCHUNKS = BPW * 2
NCHAIN = 4


def _rnn_chunk(gbuf, outb, lo, carry):
    def body_t(tl, carry):
        h0, h1 = carry
        p0 = [None] * NCHAIN
        p1 = [None] * NCHAIN
        for s in range(S):
            hs = h0[s] if s < 16 else h1[s - 16]
            c = s % NCHAIN
            m0 = hs * gbuf[tl, pl.ds(s * S, 16)]
            m1 = hs * gbuf[tl, pl.ds(s * S + 16, 16)]
            p0[c] = m0 if p0[c] is None else p0[c] + m0
            p1[c] = m1 if p1[c] is None else p1[c] + m1
        acc0 = (p0[0] + p0[1]) + (p0[2] + p0[3])
        acc1 = (p1[0] + p1[1]) + (p1[2] + p1[3])
        acc0 = jnp.clip(acc0, -10.0, 10.0)
        acc1 = jnp.clip(acc1, -10.0, 10.0)
        outb[tl, pl.ds(0, 16)] = acc0
        outb[tl, pl.ds(16, 16)] = acc1
        return (acc0, acc1)

    return lax.fori_loop(lo, CHUNK, body_t, carry)


def _sc_kernel(inp_hbm, table_hbm, out_hbm, idx_v,
               gb0, gb1, gb2, ob0, ob1, ob2,
               sg0, sg1, sg2, so0, so1, so2):
    nc = 2
    wid = lax.axis_index("s") * nc + lax.axis_index("c")
    base = wid * BPW
    gbs = [gb0, gb1, gb2]
    obs = [ob0, ob1, ob2]
    sgs = [sg0, sg1, sg2]
    sos = [so0, so1, so2]

    pltpu.sync_copy(inp_hbm.at[pl.ds(base, BPW)], idx_v)

    def start_gather(b, c, slot):
        pltpu.async_copy(
            table_hbm.at[idx_v.at[b, c, pl.ds(0, CHUNK)]],
            gbs[slot], sgs[slot])

    def wait_gather(b, c, slot):
        pltpu.make_async_copy(
            table_hbm.at[idx_v.at[b, c, pl.ds(0, CHUNK)]],
            gbs[slot], sgs[slot]).wait()

    def process(b, c, slot, carry, first):
        ob = obs[slot]
        @pl.when(jnp.logical_not(first))
        def _():
            pltpu.make_async_copy(
                ob, out_hbm.at[base, 0], sos[slot]).wait()

        wait_gather(b, c, slot)
        gb = gbs[slot]
        if c == 0:
            a0 = jnp.clip(gb[0, pl.ds(0, 16)], -10.0, 10.0)
            a1 = jnp.clip(gb[0, pl.ds(16, 16)], -10.0, 10.0)
            ob[0, pl.ds(0, 16)] = a0
            ob[0, pl.ds(16, 16)] = a1
            carry = _rnn_chunk(gb, ob, 1, (a0, a1))
        else:
            carry = _rnn_chunk(gb, ob, 0, carry)
        pltpu.async_copy(
            ob, out_hbm.at[base + b, c], sos[slot])
        return carry

    start_gather(0, 0, 0)
    start_gather(0, 1, 1)

    zero = jnp.zeros((16,), jnp.float32)

    def body(i, carry):
        j0 = i * 6
        for k in range(6):
            b = 3 * i + k // 2
            c = k % 2
            slot = k % 3
            jn = j0 + k + 2
            @pl.when(jn < NCHUNKS)
            def _():
                start_gather(3 * i + (k + 2) // 2, (k + 2) % 2,
                             (k + 2) % 3)
            carry = process(b, c, slot, carry, first=(i == 0) & (k < 3))
        return carry

    lax.fori_loop(0, NCHUNKS // 6, body, (zero, zero))

    carry = (zero, zero)
    for k in range(NCHUNKS % 6):
        j = (NCHUNKS // 6) * 6 + k
        b = j // 2
        c = j % 2
        jn = j + 2
        if jn < NCHUNKS:
            start_gather(jn // 2, jn % 2, jn % 3)
        carry = process(b, c, j % 3, carry, first=False)

    for k in range(3):
        j = NCHUNKS - 3 + k
        pltpu.make_async_copy(
            obs[j % 3], out_hbm.at[base, 0],
            sos[j % 3]).wait()


@jax.jit
def _run(inp, table2d):
    mesh = plsc.VectorSubcoreMesh(core_axis_name="c", subcore_axis_name="s")
    f = functools.partial(
        pl.kernel,
        mesh=mesh,
        out_type=jax.ShapeDtypeStruct((B, 2, CHUNK, S), jnp.float32),
        scratch_types=[
            pltpu.VMEM((BPW, 2, 32), jnp.int32),
            pltpu.VMEM((CHUNK, S * S), jnp.float32),
            pltpu.VMEM((CHUNK, S * S), jnp.float32),
            pltpu.VMEM((CHUNK, S * S), jnp.float32),
            pltpu.VMEM((CHUNK, S), jnp.float32),
            pltpu.VMEM((CHUNK, S), jnp.float32),
            pltpu.VMEM((CHUNK, S), jnp.float32),
            pltpu.SemaphoreType.DMA,
            pltpu.SemaphoreType.DMA,
            pltpu.SemaphoreType.DMA,
            pltpu.SemaphoreType.DMA,
            pltpu.SemaphoreType.DMA,
            pltpu.SemaphoreType.DMA,
        ],
    )(_sc_kernel)
    return f(inp, table2d)


def kernel(input, lengths, fsa_tensor):
    del lengths
    table2d = fsa_tensor.reshape(V, S * S)
    inp = input.astype(jnp.int32).reshape(B, 2, CHUNK)
    inp = jnp.pad(inp, ((0, 0), (0, 0), (0, 32 - CHUNK)))
    return _run(inp, table2d).reshape(B, L, S)
